# baseline (speedup 1.0000x reference)
import functools

import jax
import jax.numpy as jnp
from jax.experimental import pallas as pl
from jax.experimental.pallas import tpu as pltpu

_BLOCK_S = 8192
_EPS = 1e-5


def _fused_kernel(feat_ref, w1_ref, scale_ref, bias_ref, w2_ref, out_ref,
                  x_ref, stats_ref, affa_ref, affc_ref, *, nbatch, nb, hw):
    b = pl.program_id(0)
    sb = pl.program_id(1)
    par = jax.lax.rem(b, 2)

    @pl.when(b >= 1)
    def _stageB():
        @pl.when(sb == 0)
        def _affine():
            ri = jax.lax.broadcasted_iota(jnp.int32, (64, 64), 0)
            ci = jax.lax.broadcasted_iota(jnp.int32, (64, 64), 1)
            m = ((ri // 2) == (ci // 2)).astype(jnp.float32)
            prev = stats_ref[1 - par]
            gsum = jnp.dot(m, prev, preferred_element_type=jnp.float32)
            n = jnp.float32(2 * hw)
            mean = gsum[:, 0:1] / n
            var = gsum[:, 1:2] / n - mean * mean
            rstd = jax.lax.rsqrt(var + _EPS)
            a = scale_ref[...] * rstd
            c = bias_ref[...] - mean * a
            affa_ref[...] = jnp.broadcast_to(a, affa_ref.shape)
            affc_ref[...] = jnp.broadcast_to(c, affc_ref.shape)

        xb = x_ref[1 - par, :, pl.ds(sb * _BLOCK_S, _BLOCK_S)]
        r = jnp.maximum(affa_ref[...] * xb.astype(jnp.float32) + affc_ref[...],
                        0.0).astype(jnp.bfloat16)
        out_ref[0] = jnp.dot(w2_ref[...], r,
                             preferred_element_type=jnp.float32)

    @pl.when(b < nbatch)
    def _stageA():
        @pl.when(sb == 0)
        def _zero():
            stats_ref[par] = jnp.zeros_like(stats_ref[par])

        xb = jnp.dot(w1_ref[...], feat_ref[0],
                     preferred_element_type=jnp.float32)
        x_ref[par, :, pl.ds(sb * _BLOCK_S, _BLOCK_S)] = xb.astype(jnp.bfloat16)
        s = jnp.sum(xb, axis=1, keepdims=True)
        ss = jnp.sum(xb * xb, axis=1, keepdims=True)
        stats_ref[par, :, 0:1] += s
        stats_ref[par, :, 1:2] += ss


@jax.jit
def kernel(coarse_feat, W1, gn_scale, gn_bias, W2):
    B, C, H, W = coarse_feat.shape
    hw = H * W
    nb = hw // _BLOCK_S
    O1 = W1.shape[0]
    O2 = W2.shape[0]

    feat = coarse_feat.reshape(B, C, hw).astype(jnp.bfloat16)
    scale = gn_scale.reshape(O1, 1)
    bias = gn_bias.reshape(O1, 1)

    grid = (B + 1, nb)

    out = pl.pallas_call(
        functools.partial(_fused_kernel, nbatch=B, nb=nb, hw=hw),
        grid=grid,
        in_specs=[
            pl.BlockSpec(
                (1, C, _BLOCK_S),
                lambda b, s: (jnp.minimum(b, B - 1), 0,
                              jnp.where(b == B, nb - 1, s)),
            ),
            pl.BlockSpec((O1, C), lambda b, s: (0, 0)),
            pl.BlockSpec((O1, 1), lambda b, s: (0, 0)),
            pl.BlockSpec((O1, 1), lambda b, s: (0, 0)),
            pl.BlockSpec((O2, O1), lambda b, s: (0, 0)),
        ],
        out_specs=pl.BlockSpec(
            (1, O2, _BLOCK_S),
            lambda b, s: (jnp.maximum(b - 1, 0), 0,
                          jnp.where(b == 0, 0, s)),
        ),
        out_shape=jax.ShapeDtypeStruct((B, O2, hw), jnp.float32),
        scratch_shapes=[
            pltpu.VMEM((2, O1, hw), jnp.bfloat16),
            pltpu.VMEM((2, O1, 2), jnp.float32),
            pltpu.VMEM((O1, _BLOCK_S), jnp.float32),
            pltpu.VMEM((O1, _BLOCK_S), jnp.float32),
        ],
    )(feat, W1.astype(jnp.bfloat16), scale, bias, W2.astype(jnp.bfloat16))

    return out.reshape(B, O2, H, W)


# speedup vs baseline: 1.1266x; 1.1266x over previous
import functools

import jax
import jax.numpy as jnp
from jax.experimental import pallas as pl
from jax.experimental.pallas import tpu as pltpu

_BLOCK_S = 8192
_EPS = 1e-5


def _fused_kernel(feat_ref, w1_ref, scale_ref, bias_ref, w2_ref, out_ref,
                  x_ref, stats_ref, affa_ref, affc_ref, *, nbatch, nb, hw):
    b = pl.program_id(0)
    sb = pl.program_id(1)
    par = jax.lax.rem(b, 2)

    @pl.when(b >= 1)
    def _stageB():
        @pl.when(sb == 0)
        def _affine():
            ri = jax.lax.broadcasted_iota(jnp.int32, (64, 64), 0)
            ci = jax.lax.broadcasted_iota(jnp.int32, (64, 64), 1)
            m = ((ri // 2) == (ci // 2)).astype(jnp.float32)
            prev = stats_ref[1 - par]
            gsum = jnp.dot(m, prev, preferred_element_type=jnp.float32)
            n = jnp.float32(2 * hw)
            mean = gsum[:, 0:1] / n
            var = gsum[:, 1:2] / n - mean * mean
            rstd = jax.lax.rsqrt(var + _EPS)
            a = scale_ref[...] * rstd
            c = bias_ref[...] - mean * a
            affa_ref[...] = jnp.broadcast_to(a, affa_ref.shape)
            affc_ref[...] = jnp.broadcast_to(c, affc_ref.shape)

        xb = x_ref[1 - par, :, pl.ds(sb * _BLOCK_S, _BLOCK_S)]
        r = jnp.maximum(affa_ref[...] * xb.astype(jnp.float32) + affc_ref[...],
                        0.0).astype(jnp.bfloat16)
        out_ref[0] = jnp.dot(w2_ref[...], r,
                             preferred_element_type=jnp.float32)

    @pl.when(b < nbatch)
    def _stageA():
        @pl.when(sb == 0)
        def _zero():
            stats_ref[par] = jnp.zeros_like(stats_ref[par])

        fb = feat_ref[0].astype(jnp.bfloat16)
        xb = jnp.dot(w1_ref[...], fb,
                     preferred_element_type=jnp.float32)
        x_ref[par, :, pl.ds(sb * _BLOCK_S, _BLOCK_S)] = xb.astype(jnp.bfloat16)
        s = jnp.sum(xb, axis=1, keepdims=True)
        ss = jnp.sum(xb * xb, axis=1, keepdims=True)
        stats_ref[par, :, 0:1] += s
        stats_ref[par, :, 1:2] += ss


@jax.jit
def kernel(coarse_feat, W1, gn_scale, gn_bias, W2):
    B, C, H, W = coarse_feat.shape
    hw = H * W
    nb = hw // _BLOCK_S
    O1 = W1.shape[0]
    O2 = W2.shape[0]

    feat = coarse_feat.reshape(B, C, hw)
    scale = gn_scale.reshape(O1, 1)
    bias = gn_bias.reshape(O1, 1)

    grid = (B + 1, nb)

    out = pl.pallas_call(
        functools.partial(_fused_kernel, nbatch=B, nb=nb, hw=hw),
        grid=grid,
        in_specs=[
            pl.BlockSpec(
                (1, C, _BLOCK_S),
                lambda b, s: (jnp.minimum(b, B - 1), 0,
                              jnp.where(b == B, nb - 1, s)),
            ),
            pl.BlockSpec((O1, C), lambda b, s: (0, 0)),
            pl.BlockSpec((O1, 1), lambda b, s: (0, 0)),
            pl.BlockSpec((O1, 1), lambda b, s: (0, 0)),
            pl.BlockSpec((O2, O1), lambda b, s: (0, 0)),
        ],
        out_specs=pl.BlockSpec(
            (1, O2, _BLOCK_S),
            lambda b, s: (jnp.maximum(b - 1, 0), 0,
                          jnp.where(b == 0, 0, s)),
        ),
        out_shape=jax.ShapeDtypeStruct((B, O2, hw), jnp.float32),
        scratch_shapes=[
            pltpu.VMEM((2, O1, hw), jnp.bfloat16),
            pltpu.VMEM((2, O1, 2), jnp.float32),
            pltpu.VMEM((O1, _BLOCK_S), jnp.float32),
            pltpu.VMEM((O1, _BLOCK_S), jnp.float32),
        ],
    )(feat, W1.astype(jnp.bfloat16), scale, bias, W2.astype(jnp.bfloat16))

    return out.reshape(B, O2, H, W)


# speedup vs baseline: 1.1853x; 1.0521x over previous
import functools

import jax
import jax.numpy as jnp
from jax.experimental import pallas as pl
from jax.experimental.pallas import tpu as pltpu

_BLOCK_S = 16384
_EPS = 1e-5


def _fused_kernel(feat_ref, w1_ref, scale_ref, bias_ref, w2_ref, out_ref,
                  x_ref, stats_ref, affa_ref, affc_ref, *, nbatch, nb, hw):
    b = pl.program_id(0)
    sb = pl.program_id(1)
    par = jax.lax.rem(b, 2)

    @pl.when(b >= 1)
    def _stageB():
        @pl.when(sb == 0)
        def _affine():
            ri = jax.lax.broadcasted_iota(jnp.int32, (64, 64), 0)
            ci = jax.lax.broadcasted_iota(jnp.int32, (64, 64), 1)
            m = ((ri // 2) == (ci // 2)).astype(jnp.float32)
            prev = stats_ref[1 - par]
            gsum = jnp.dot(m, prev, preferred_element_type=jnp.float32)
            n = jnp.float32(2 * hw)
            mean = gsum[:, 0:1] / n
            var = gsum[:, 1:2] / n - mean * mean
            rstd = jax.lax.rsqrt(var + _EPS)
            a = scale_ref[...] * rstd
            c = bias_ref[...] - mean * a
            affa_ref[...] = jnp.broadcast_to(a, affa_ref.shape)
            affc_ref[...] = jnp.broadcast_to(c, affc_ref.shape)

        xb = x_ref[1 - par, :, pl.ds(sb * _BLOCK_S, _BLOCK_S)]
        r = jnp.maximum(affa_ref[...] * xb.astype(jnp.float32) + affc_ref[...],
                        0.0).astype(jnp.bfloat16)
        out_ref[0] = jnp.dot(w2_ref[...], r,
                             preferred_element_type=jnp.float32)

    @pl.when(b < nbatch)
    def _stageA():
        @pl.when(sb == 0)
        def _zero():
            stats_ref[par] = jnp.zeros_like(stats_ref[par])

        fb = feat_ref[0].astype(jnp.bfloat16)
        xb = jnp.dot(w1_ref[...], fb,
                     preferred_element_type=jnp.float32)
        x_ref[par, :, pl.ds(sb * _BLOCK_S, _BLOCK_S)] = xb.astype(jnp.bfloat16)
        s = jnp.sum(xb, axis=1, keepdims=True)
        ss = jnp.sum(xb * xb, axis=1, keepdims=True)
        stats_ref[par, :, 0:1] += s
        stats_ref[par, :, 1:2] += ss


@jax.jit
def kernel(coarse_feat, W1, gn_scale, gn_bias, W2):
    B, C, H, W = coarse_feat.shape
    hw = H * W
    nb = hw // _BLOCK_S
    O1 = W1.shape[0]
    O2 = W2.shape[0]

    feat = coarse_feat.reshape(B, C, hw)
    scale = gn_scale.reshape(O1, 1)
    bias = gn_bias.reshape(O1, 1)

    grid = (B + 1, nb)

    out = pl.pallas_call(
        functools.partial(_fused_kernel, nbatch=B, nb=nb, hw=hw),
        grid=grid,
        in_specs=[
            pl.BlockSpec(
                (1, C, _BLOCK_S),
                lambda b, s: (jnp.minimum(b, B - 1), 0,
                              jnp.where(b == B, nb - 1, s)),
            ),
            pl.BlockSpec((O1, C), lambda b, s: (0, 0)),
            pl.BlockSpec((O1, 1), lambda b, s: (0, 0)),
            pl.BlockSpec((O1, 1), lambda b, s: (0, 0)),
            pl.BlockSpec((O2, O1), lambda b, s: (0, 0)),
        ],
        out_specs=pl.BlockSpec(
            (1, O2, _BLOCK_S),
            lambda b, s: (jnp.maximum(b - 1, 0), 0,
                          jnp.where(b == 0, 0, s)),
        ),
        out_shape=jax.ShapeDtypeStruct((B, O2, hw), jnp.float32),
        scratch_shapes=[
            pltpu.VMEM((2, O1, hw), jnp.bfloat16),
            pltpu.VMEM((2, O1, 2), jnp.float32),
            pltpu.VMEM((O1, _BLOCK_S), jnp.float32),
            pltpu.VMEM((O1, _BLOCK_S), jnp.float32),
        ],
    )(feat, W1.astype(jnp.bfloat16), scale, bias, W2.astype(jnp.bfloat16))

    return out.reshape(B, O2, H, W)
